# baseline (speedup 1.0000x reference)
import functools

import jax
import jax.numpy as jnp
from jax import lax
from jax.experimental import pallas as pl
from jax.experimental.pallas import tpu as pltpu
from jax.experimental.pallas import tpu_sc as plsc

NC = 2
NS = 16
L = 16
NW = NC * NS

B = 16384
BPW = B // NW
TAB = 1000

_mesh = plsc.VectorSubcoreMesh(
    core_axis_name="c", subcore_axis_name="s"
)


@functools.partial(
    pl.kernel,
    mesh=_mesh,
    out_type=jax.ShapeDtypeStruct((B,), jnp.float32),
    scratch_types=[
        pltpu.VMEM((TAB,), jnp.float32),
        pltpu.VMEM((BPW,), jnp.int32),
        pltpu.VMEM((BPW,), jnp.float32),
        pltpu.SemaphoreType.DMA,
    ],
    compiler_params=pltpu.CompilerParams(
        needs_layout_passes=False,
        skip_device_barrier=True,
        disable_bounds_checks=True,
        disable_semaphore_checks=True,
    ),
)
def _gather_kernel(gam_hbm, t_hbm, out_hbm, gam_v, idx_v, out_v, sem):
    base = (lax.axis_index("s") * NC + lax.axis_index("c")) * BPW
    cp_g = pltpu.async_copy(gam_hbm.at[pl.ds(0, TAB)], gam_v, sem)
    cp_t = pltpu.async_copy(t_hbm.at[pl.ds(base, BPW)], idx_v, sem)
    cp_g.wait()
    cp_t.wait()

    @plsc.parallel_loop(0, BPW // L, unroll=4)
    def body(i):
        o = i * L
        idx = idx_v[pl.ds(o, L)]
        out_v[pl.ds(o, L)] = plsc.load_gather(gam_v, [idx])

    pltpu.async_copy(out_v, out_hbm.at[pl.ds(base, BPW)], sem).wait()


def kernel(t, gammas):
    return _gather_kernel(gammas.astype(jnp.float32), t.astype(jnp.int32))


# speedup vs baseline: 1.0740x; 1.0740x over previous
import functools

import jax
import jax.numpy as jnp
from jax import lax
from jax.experimental import pallas as pl
from jax.experimental.pallas import tpu as pltpu
from jax.experimental.pallas import tpu_sc as plsc

NC = 1
NS = 16
L = 16
NW = NC * NS

B = 16384
BPW = B // NW
TAB = 1000

_mesh = plsc.VectorSubcoreMesh(
    core_axis_name="c", subcore_axis_name="s", num_cores=NC
)


@functools.partial(
    pl.kernel,
    mesh=_mesh,
    out_type=jax.ShapeDtypeStruct((B,), jnp.float32),
    scratch_types=[
        pltpu.VMEM((TAB,), jnp.float32),
        pltpu.VMEM((BPW,), jnp.int32),
        pltpu.VMEM((BPW,), jnp.float32),
        pltpu.SemaphoreType.DMA,
    ],
    compiler_params=pltpu.CompilerParams(
        needs_layout_passes=False,
        skip_device_barrier=True,
        disable_bounds_checks=True,
        disable_semaphore_checks=True,
    ),
)
def _gather_kernel(gam_hbm, t_hbm, out_hbm, gam_v, idx_v, out_v, sem):
    base = lax.axis_index("s") * BPW
    cp_g = pltpu.async_copy(gam_hbm.at[pl.ds(0, TAB)], gam_v, sem)
    cp_t = pltpu.async_copy(t_hbm.at[pl.ds(base, BPW)], idx_v, sem)
    cp_g.wait()
    cp_t.wait()

    @plsc.parallel_loop(0, BPW // L, unroll=4)
    def body(i):
        o = i * L
        idx = idx_v[pl.ds(o, L)]
        out_v[pl.ds(o, L)] = plsc.load_gather(gam_v, [idx])

    pltpu.async_copy(out_v, out_hbm.at[pl.ds(base, BPW)], sem).wait()


def kernel(t, gammas):
    return _gather_kernel(gammas.astype(jnp.float32), t.astype(jnp.int32))
